# baseline (speedup 1.0000x reference)
import functools

import jax
import jax.numpy as jnp
from jax.experimental import pallas as pl
from jax.experimental.pallas import tpu as pltpu


def _layer_kernel(num_adjs, adj_ref, h_ref, w_ref, b_ref, o_ref):
    h = h_ref[...]
    acc = None
    for a in range(num_adjs):
        agg = jnp.dot(adj_ref[a].astype(jnp.bfloat16), h,
                      preferred_element_type=jnp.float32)
        part = jnp.dot(agg.astype(jnp.bfloat16),
                       w_ref[a].astype(jnp.bfloat16),
                       preferred_element_type=jnp.float32)
        acc = part if acc is None else acc + part
    o_ref[...] = jnp.maximum(acc + b_ref[...], 0.0).astype(o_ref.dtype)


def _layer(adjs, h_bf, w, b_v, row_tile, out_dtype):
    A, N, _ = adjs.shape
    F = h_bf.shape[1]
    H = w.shape[2]
    kern = functools.partial(_layer_kernel, A)
    return pl.pallas_call(
        kern,
        out_shape=jax.ShapeDtypeStruct((N, H), out_dtype),
        grid=(N // row_tile,),
        in_specs=[
            pl.BlockSpec((A, row_tile, N), lambda i: (0, i, 0)),
            pl.BlockSpec((N, F), lambda i: (0, 0)),
            pl.BlockSpec((A, F, H), lambda i: (0, 0, 0)),
            pl.BlockSpec((1, H), lambda i: (0, 0)),
        ],
        out_specs=pl.BlockSpec((row_tile, H), lambda i: (i, 0)),
        compiler_params=pltpu.CompilerParams(
            dimension_semantics=("parallel",),
            vmem_limit_bytes=int(0.9 * 64 * 1024 * 1024)),
    )(adjs, h_bf, w, b_v)


def kernel(x, adjs, w1, b1, w2, b2):
    N, _ = x.shape
    H = w1.shape[2]
    out_dtype = x.dtype

    row_tile = 128 if N % 128 == 0 else 8

    x_bf = x.astype(jnp.bfloat16)
    b1_v = b1.astype(jnp.float32).reshape(1, H)
    b2_v = b2.astype(jnp.float32).reshape(1, H)

    x1 = _layer(adjs, x_bf, w1.astype(jnp.float32), b1_v, row_tile,
                jnp.bfloat16)
    out = _layer(adjs, x1, w2.astype(jnp.float32), b2_v, row_tile, out_dtype)
    return out


# speedup vs baseline: 1.2043x; 1.2043x over previous
import functools

import jax
import jax.numpy as jnp
from jax.experimental import pallas as pl
from jax.experimental.pallas import tpu as pltpu


def _layer_kernel(num_adjs, adj_ref, h_ref, w_ref, b_ref, o_ref):
    h = h_ref[...].astype(jnp.bfloat16)
    acc = None
    for a in range(num_adjs):
        agg = jnp.dot(adj_ref[a].astype(jnp.bfloat16), h,
                      preferred_element_type=jnp.float32)
        part = jnp.dot(agg.astype(jnp.bfloat16), w_ref[a],
                       preferred_element_type=jnp.float32)
        acc = part if acc is None else acc + part
    o_ref[...] = jnp.maximum(acc + b_ref[...], 0.0).astype(o_ref.dtype)


def _layer(adjs, h, w, b_v, row_tile, out_dtype):
    A, N, _ = adjs.shape
    F = h.shape[1]
    H = w.shape[2]
    kern = functools.partial(_layer_kernel, A)
    return pl.pallas_call(
        kern,
        out_shape=jax.ShapeDtypeStruct((N, H), out_dtype),
        grid=(N // row_tile,),
        in_specs=[
            pl.BlockSpec((A, row_tile, N), lambda i: (0, i, 0)),
            pl.BlockSpec((N, F), lambda i: (0, 0)),
            pl.BlockSpec((A, F, H), lambda i: (0, 0, 0)),
            pl.BlockSpec((1, H), lambda i: (0, 0)),
        ],
        out_specs=pl.BlockSpec((row_tile, H), lambda i: (i, 0)),
        compiler_params=pltpu.CompilerParams(
            dimension_semantics=("parallel",),
            vmem_limit_bytes=int(0.9 * 64 * 1024 * 1024)),
    )(adjs, h, w, b_v)


def kernel(x, adjs, w1, b1, w2, b2):
    N, _ = x.shape
    H = w1.shape[2]
    out_dtype = x.dtype

    row_tile = 256 if N % 256 == 0 else 8

    b1_v = b1.astype(jnp.float32).reshape(1, H)
    b2_v = b2.astype(jnp.float32).reshape(1, H)

    x1 = _layer(adjs, x, w1.astype(jnp.bfloat16), b1_v, row_tile,
                jnp.bfloat16)
    out = _layer(adjs, x1, w2.astype(jnp.bfloat16), b2_v, row_tile, out_dtype)
    return out
